# baseline (speedup 1.0000x reference)
import functools

import jax
import jax.numpy as jnp
from jax import lax
from jax.experimental import pallas as pl
from jax.experimental.pallas import tpu as pltpu
from jax.experimental.pallas import tpu_sc as plsc

NC = 2
NS = 16
NW = NC * NS
LANES = 16
NBUF = 4
EPS = 1e-12


def _fast_rsqrt(x):
    i = plsc.bitcast(x, jnp.int32)
    i = jnp.int32(0x5F3759DF) - lax.shift_right_logical(i, 1)
    y = plsc.bitcast(i, jnp.float32)
    for _ in range(2):
        y = y * (1.5 - 0.5 * x * y * y)
    return y


def _make_sc_kernel(n_rows, embed, c_rows, n_iter):
    mesh = plsc.VectorSubcoreMesh(
        core_axis_name="c", subcore_axis_name="s", num_cores=NC, num_subcores=NS
    )

    @functools.partial(
        pl.kernel,
        mesh=mesh,
        out_type=jax.ShapeDtypeStruct((n_rows, embed), jnp.float32),
        compiler_params=pltpu.CompilerParams(
            needs_layout_passes=False, use_tc_tiling_on_sc=False
        ),
        scratch_types=[
            [[pltpu.VMEM((c_rows // 2,), jnp.int32) for _ in range(2)]
             for _ in range(NBUF)],
            [pltpu.VMEM((c_rows, embed), jnp.float32) for _ in range(NBUF)],
            pltpu.VMEM((embed,), jnp.float32),
            pltpu.VMEM((embed,), jnp.float32),
            [pltpu.SemaphoreType.DMA for _ in range(NBUF)],
            [pltpu.SemaphoreType.DMA for _ in range(NBUF)],
            [pltpu.SemaphoreType.DMA for _ in range(NBUF)],
        ],
    )
    def body(ids_hbm, table_hbm, gamma_hbm, beta_hbm, out_hbm,
             idxbs, bufs, gv, bv, sxs, sis, sos):
        w = lax.axis_index("s") * NC + lax.axis_index("c")
        pltpu.sync_copy(gamma_hbm, gv)
        pltpu.sync_copy(beta_hbm, bv)

        lane = lax.iota(jnp.int32, LANES)
        inv_e = jnp.float32(1.0 / embed)
        n_groups = c_rows // LANES

        perms = [lane ^ (1 << t) for t in range(4)]

        def xlsum(v):
            for p in perms:
                v = v + jnp.take_along_axis(
                    v, p, axis=0, mode="promise_in_bounds"
                )
            return v

        nq = embed // LANES

        def compute(buf):
            gvecs = [gv[pl.ds(k * LANES, LANES)] for k in range(nq)]
            bvecs = [bv[pl.ds(k * LANES, LANES)] for k in range(nq)]

            @plsc.parallel_loop(0, c_rows, unroll=4)
            def ln_row(r):
                xs = [buf[r, pl.ds(k * LANES, LANES)] for k in range(nq)]
                s = xs[0] + xs[1] + xs[2] + xs[3]
                sq = (xs[0] * xs[0] + xs[1] * xs[1]
                      + xs[2] * xs[2] + xs[3] * xs[3])
                mean = xlsum(s) * inv_e
                var = jnp.maximum(xlsum(sq) * inv_e - mean * mean, 0.0)
                rstd = _fast_rsqrt(var + EPS)
                m2 = mean * rstd
                for k in range(nq):
                    y = (xs[k] * rstd - m2) * gvecs[k] + bvecs[k]
                    buf[r, pl.ds(k * LANES, LANES)] = y

        half = c_rows // 2

        def start_idx(i, b):
            pltpu.async_copy(ids_hbm.at[w, i, pl.ds(0, half)], idxbs[b][0],
                             sxs[b])
            pltpu.async_copy(ids_hbm.at[w, i, pl.ds(half, half)], idxbs[b][1],
                             sxs[b])

        def wait_idx(i, b):
            pltpu.make_async_copy(ids_hbm.at[w, i, pl.ds(0, half)],
                                  idxbs[b][0], sxs[b]).wait()
            pltpu.make_async_copy(ids_hbm.at[w, i, pl.ds(half, half)],
                                  idxbs[b][1], sxs[b]).wait()

        def start_in(b):
            pltpu.async_copy(table_hbm.at[idxbs[b][0]],
                             bufs[b].at[pl.ds(0, half)], sis[b])
            pltpu.async_copy(table_hbm.at[idxbs[b][1]],
                             bufs[b].at[pl.ds(half, half)], sis[b])

        def wait_in(b):
            pltpu.make_async_copy(table_hbm.at[idxbs[b][0]],
                                  bufs[b].at[pl.ds(0, half)], sis[b]).wait()
            pltpu.make_async_copy(table_hbm.at[idxbs[b][1]],
                                  bufs[b].at[pl.ds(half, half)], sis[b]).wait()

        def out_slice(i):
            return out_hbm.at[pl.ds((w * n_iter + i) * c_rows, c_rows)]

        start_idx(0, 0)
        start_idx(1, 1)
        start_idx(2, 2)
        wait_idx(0, 0)
        start_in(0)
        wait_idx(1, 1)
        start_in(1)

        def step(i, b):
            wait_in(b)

            b3 = (b + 3) % NBUF

            @pl.when(i + 3 < n_iter)
            def _():
                start_idx(i + 3, b3)

            compute(bufs[b])
            pltpu.async_copy(bufs[b], out_slice(i), sos[b])

            j = i + 2
            b2 = (b + 2) % NBUF

            @pl.when(j < n_iter)
            def _():
                @pl.when(j >= NBUF)
                def _():
                    pltpu.make_async_copy(
                        bufs[b2], out_slice(j - NBUF), sos[b2]
                    ).wait()

                wait_idx(j, b2)
                start_in(b2)

        def outer(o, _):
            for b in range(NBUF):
                step(o * NBUF + b, b)
            return 0

        lax.fori_loop(0, n_iter // NBUF, outer, 0)
        for b in range(NBUF):
            i = n_iter - NBUF + b
            pltpu.make_async_copy(bufs[b], out_slice(i), sos[b]).wait()

    return body


def kernel(input_ids, table, gamma, beta):
    b, s = input_ids.shape
    vocab, embed = table.shape
    n = b * s
    c_rows = 256
    n_iter = n // (NW * c_rows)
    assert n == NW * n_iter * c_rows and n_iter % NBUF == 0

    ids = input_ids.reshape(NW, n_iter, c_rows).astype(jnp.int32)
    sc = _make_sc_kernel(n, embed, c_rows, n_iter)
    out = sc(ids, table, gamma, beta)
    return out.reshape(b, s, embed)


# speedup vs baseline: 1.0015x; 1.0015x over previous
import functools

import jax
import jax.numpy as jnp
from jax import lax
from jax.experimental import pallas as pl
from jax.experimental.pallas import tpu as pltpu
from jax.experimental.pallas import tpu_sc as plsc

NC = 2
NS = 16
NW = NC * NS
LANES = 16
NBUF = 4
BPC = 2
EPS = 1e-12


def _fast_rsqrt(x):
    i = plsc.bitcast(x, jnp.int32)
    i = jnp.int32(0x5F3759DF) - lax.shift_right_logical(i, 1)
    y = plsc.bitcast(i, jnp.float32)
    for _ in range(2):
        y = y * (1.5 - 0.5 * x * y * y)
    return y


def _make_sc_kernel(n_batch, seq, embed):
    mesh = plsc.VectorSubcoreMesh(
        core_axis_name="c", subcore_axis_name="s", num_cores=NC, num_subcores=NS
    )
    c_rows = BPC * seq
    n_iter = n_batch // (NW * BPC)

    @functools.partial(
        pl.kernel,
        mesh=mesh,
        out_type=jax.ShapeDtypeStruct((n_batch, seq, embed), jnp.float32),
        compiler_params=pltpu.CompilerParams(
            needs_layout_passes=False, use_tc_tiling_on_sc=False
        ),
        scratch_types=[
            [pltpu.VMEM((c_rows,), jnp.int32) for _ in range(NBUF)],
            [pltpu.VMEM((c_rows, embed), jnp.float32) for _ in range(NBUF)],
            pltpu.VMEM((embed,), jnp.float32),
            pltpu.VMEM((embed,), jnp.float32),
            [pltpu.SemaphoreType.DMA for _ in range(NBUF)],
            [pltpu.SemaphoreType.DMA for _ in range(NBUF)],
            [pltpu.SemaphoreType.DMA for _ in range(NBUF)],
        ],
    )
    def body(ids_hbm, table_hbm, gamma_hbm, beta_hbm, out_hbm,
             idxbs, bufs, gv, bv, sxs, sis, sos):
        w = lax.axis_index("s") * NC + lax.axis_index("c")
        pltpu.sync_copy(gamma_hbm, gv)
        pltpu.sync_copy(beta_hbm, bv)

        lane = lax.iota(jnp.int32, LANES)
        inv_e = jnp.float32(1.0 / embed)

        perms = [lane ^ (1 << t) for t in range(4)]

        def xlsum(v):
            for p in perms:
                v = v + jnp.take_along_axis(
                    v, p, axis=0, mode="promise_in_bounds"
                )
            return v

        nq = embed // LANES

        def compute(buf):
            gvecs = [gv[pl.ds(k * LANES, LANES)] for k in range(nq)]
            bvecs = [bv[pl.ds(k * LANES, LANES)] for k in range(nq)]

            @plsc.parallel_loop(0, c_rows, unroll=4)
            def ln_row(r):
                xs = [buf[r, pl.ds(k * LANES, LANES)] for k in range(nq)]
                s = xs[0] + xs[1] + xs[2] + xs[3]
                sq = (xs[0] * xs[0] + xs[1] * xs[1]
                      + xs[2] * xs[2] + xs[3] * xs[3])
                mean = xlsum(s) * inv_e
                var = jnp.maximum(xlsum(sq) * inv_e - mean * mean, 0.0)
                rstd = _fast_rsqrt(var + EPS)
                m2 = mean * rstd
                for k in range(nq):
                    y = (xs[k] * rstd - m2) * gvecs[k] + bvecs[k]
                    buf[r, pl.ds(k * LANES, LANES)] = y

        def batch0(i):
            return (w * n_iter + i) * BPC

        def start_idx(i, b):
            for j in range(BPC):
                pltpu.async_copy(ids_hbm.at[batch0(i) + j],
                                 idxbs[b].at[pl.ds(j * seq, seq)], sxs[b])

        def wait_idx(i, b):
            for j in range(BPC):
                pltpu.make_async_copy(
                    ids_hbm.at[batch0(i) + j],
                    idxbs[b].at[pl.ds(j * seq, seq)], sxs[b]
                ).wait()

        def start_in(b):
            pltpu.async_copy(table_hbm.at[idxbs[b]], bufs[b], sis[b])

        def wait_in(b):
            pltpu.make_async_copy(table_hbm.at[idxbs[b]], bufs[b], sis[b]).wait()

        def start_out(i, b):
            for j in range(BPC):
                pltpu.async_copy(bufs[b].at[pl.ds(j * seq, seq)],
                                 out_hbm.at[batch0(i) + j], sos[b])

        def wait_out(i, b):
            for j in range(BPC):
                pltpu.make_async_copy(
                    bufs[b].at[pl.ds(j * seq, seq)],
                    out_hbm.at[batch0(i) + j], sos[b]
                ).wait()

        start_idx(0, 0)
        start_idx(1, 1)
        start_idx(2, 2)
        wait_idx(0, 0)
        start_in(0)
        wait_idx(1, 1)
        start_in(1)

        def step(i, b):
            wait_in(b)

            b3 = (b + 3) % NBUF

            @pl.when(i + 3 < n_iter)
            def _():
                start_idx(i + 3, b3)

            compute(bufs[b])
            start_out(i, b)

            j = i + 2
            b2 = (b + 2) % NBUF

            @pl.when(j < n_iter)
            def _():
                @pl.when(j >= NBUF)
                def _():
                    wait_out(j - NBUF, b2)

                wait_idx(j, b2)
                start_in(b2)

        def outer(o, _):
            for b in range(NBUF):
                step(o * NBUF + b, b)
            return 0

        lax.fori_loop(0, n_iter // NBUF, outer, 0)
        for b in range(NBUF):
            wait_out(n_iter - NBUF + b, b)

    return body


def kernel(input_ids, table, gamma, beta):
    n_batch, seq = input_ids.shape
    vocab, embed = table.shape
    assert n_batch % (NW * BPC) == 0 and embed % LANES == 0

    sc = _make_sc_kernel(n_batch, seq, embed)
    return sc(input_ids.astype(jnp.int32), table, gamma, beta)


# speedup vs baseline: 1.0153x; 1.0138x over previous
import functools

import jax
import jax.numpy as jnp
from jax import lax
from jax.experimental import pallas as pl
from jax.experimental.pallas import tpu as pltpu
from jax.experimental.pallas import tpu_sc as plsc

NC = 2
NS = 16
NW = NC * NS
LANES = 16
NBUF = 4
BPC = 2
EPS = 1e-12


def _fast_rsqrt(x):
    i = plsc.bitcast(x, jnp.int32)
    i = jnp.int32(0x5F3759DF) - lax.shift_right_logical(i, 1)
    y = plsc.bitcast(i, jnp.float32)
    for _ in range(1):
        y = y * (1.5 - 0.5 * x * y * y)
    return y


def _make_sc_kernel(n_batch, seq, embed):
    mesh = plsc.VectorSubcoreMesh(
        core_axis_name="c", subcore_axis_name="s", num_cores=NC, num_subcores=NS
    )
    c_rows = BPC * seq
    n_iter = n_batch // (NW * BPC)

    @functools.partial(
        pl.kernel,
        mesh=mesh,
        out_type=jax.ShapeDtypeStruct((n_batch, seq, embed), jnp.float32),
        compiler_params=pltpu.CompilerParams(
            needs_layout_passes=False, use_tc_tiling_on_sc=False
        ),
        scratch_types=[
            [pltpu.VMEM((c_rows,), jnp.int32) for _ in range(NBUF)],
            [pltpu.VMEM((c_rows, embed), jnp.float32) for _ in range(NBUF)],
            pltpu.VMEM((embed,), jnp.float32),
            pltpu.VMEM((embed,), jnp.float32),
            [pltpu.SemaphoreType.DMA for _ in range(NBUF)],
            [pltpu.SemaphoreType.DMA for _ in range(NBUF)],
            [pltpu.SemaphoreType.DMA for _ in range(NBUF)],
        ],
    )
    def body(ids_hbm, table_hbm, gamma_hbm, beta_hbm, out_hbm,
             idxbs, bufs, gv, bv, sxs, sis, sos):
        w = lax.axis_index("s") * NC + lax.axis_index("c")
        pltpu.sync_copy(gamma_hbm, gv)
        pltpu.sync_copy(beta_hbm, bv)

        lane = lax.iota(jnp.int32, LANES)
        inv_e = jnp.float32(1.0 / embed)

        perms = [lane ^ (1 << t) for t in range(4)]

        def xlsum(v):
            for p in perms:
                v = v + jnp.take_along_axis(
                    v, p, axis=0, mode="promise_in_bounds"
                )
            return v

        nq = embed // LANES

        def compute(buf):
            gvecs = [gv[pl.ds(k * LANES, LANES)] for k in range(nq)]
            bvecs = [bv[pl.ds(k * LANES, LANES)] for k in range(nq)]

            @plsc.parallel_loop(0, c_rows, unroll=4)
            def ln_row(r):
                xs = [buf[r, pl.ds(k * LANES, LANES)] for k in range(nq)]
                s = xs[0] + xs[1] + xs[2] + xs[3]
                sq = (xs[0] * xs[0] + xs[1] * xs[1]
                      + xs[2] * xs[2] + xs[3] * xs[3])
                mean = xlsum(s) * inv_e
                var = jnp.maximum(xlsum(sq) * inv_e - mean * mean, 0.0)
                rstd = _fast_rsqrt(var + EPS)
                m2 = mean * rstd
                for k in range(nq):
                    y = (xs[k] * rstd - m2) * gvecs[k] + bvecs[k]
                    buf[r, pl.ds(k * LANES, LANES)] = y

        def batch0(i):
            return (w * n_iter + i) * BPC

        def start_idx(i, b):
            pltpu.async_copy(ids_hbm.at[pl.ds(batch0(i) * seq, c_rows)],
                             idxbs[b], sxs[b])

        def wait_idx(i, b):
            pltpu.make_async_copy(
                ids_hbm.at[pl.ds(batch0(i) * seq, c_rows)], idxbs[b], sxs[b]
            ).wait()

        def start_in(b):
            pltpu.async_copy(table_hbm.at[idxbs[b]], bufs[b], sis[b])

        def wait_in(b):
            pltpu.make_async_copy(table_hbm.at[idxbs[b]], bufs[b], sis[b]).wait()

        def start_out(i, b):
            for j in range(BPC):
                pltpu.async_copy(bufs[b].at[pl.ds(j * seq, seq)],
                                 out_hbm.at[batch0(i) + j], sos[b])

        def wait_out(i, b):
            for j in range(BPC):
                pltpu.make_async_copy(
                    bufs[b].at[pl.ds(j * seq, seq)],
                    out_hbm.at[batch0(i) + j], sos[b]
                ).wait()

        start_idx(0, 0)
        start_idx(1, 1)
        start_idx(2, 2)
        wait_idx(0, 0)
        start_in(0)
        wait_idx(1, 1)
        start_in(1)

        def step(i, b):
            wait_in(b)

            b3 = (b + 3) % NBUF

            @pl.when(i + 3 < n_iter)
            def _():
                start_idx(i + 3, b3)

            compute(bufs[b])
            start_out(i, b)

            j = i + 2
            b2 = (b + 2) % NBUF

            @pl.when(j < n_iter)
            def _():
                @pl.when(j >= NBUF)
                def _():
                    wait_out(j - NBUF, b2)

                wait_idx(j, b2)
                start_in(b2)

        def outer(o, _):
            for b in range(NBUF):
                step(o * NBUF + b, b)
            return 0

        lax.fori_loop(0, n_iter // NBUF, outer, 0)
        for b in range(NBUF):
            wait_out(n_iter - NBUF + b, b)

    return body


def kernel(input_ids, table, gamma, beta):
    n_batch, seq = input_ids.shape
    vocab, embed = table.shape
    assert n_batch % (NW * BPC) == 0 and embed % LANES == 0

    sc = _make_sc_kernel(n_batch, seq, embed)
    ids = input_ids.astype(jnp.int32).reshape(-1)
    return sc(ids, table, gamma, beta)
